# initial kernel scaffold (unmeasured)
import jax
import jax.numpy as jnp
from jax import lax
from jax.experimental import pallas as pl
from jax.experimental.pallas import tpu as pltpu

B = 4
S = 1024
H_SHARD = 16
D = 128
K = H_SHARD * D
N = 4096
S_HALF = S // 2


def kernel(O, Wo):
    o = O.reshape(B, S, K).astype(jnp.bfloat16)
    w = Wo.astype(jnp.bfloat16)

    def body(o_ref, w_ref, out_ref, send_buf, recv_buf, send_sem, recv_sem):
        my_x = lax.axis_index("x")
        my_y = lax.axis_index("y")
        my_z = lax.axis_index("z")
        partner = (my_x, my_y, 1 - my_z)

        barrier_sem = pltpu.get_barrier_semaphore()
        pl.semaphore_signal(
            barrier_sem, inc=1,
            device_id=partner, device_id_type=pl.DeviceIdType.MESH,
        )
        pl.semaphore_wait(barrier_sem, 1)

        send_off = (1 - my_z) * S_HALF
        keep_off = my_z * S_HALF

        for b in range(B):
            p = jnp.dot(
                o_ref[b, pl.ds(send_off, S_HALF), :], w_ref[...],
                preferred_element_type=jnp.float32,
            )
            send_buf[b, :, :] = p.astype(jnp.bfloat16)

        rdma = pltpu.make_async_remote_copy(
            src_ref=send_buf,
            dst_ref=recv_buf,
            send_sem=send_sem,
            recv_sem=recv_sem,
            device_id=partner,
            device_id_type=pl.DeviceIdType.MESH,
        )
        rdma.start()

        for b in range(B):
            out_ref[b, :, :] = jnp.dot(
                o_ref[b, pl.ds(keep_off, S_HALF), :], w_ref[...],
                preferred_element_type=jnp.float32,
            )

        rdma.wait()

        for b in range(B):
            out_ref[b, :, :] = out_ref[b, :, :] + recv_buf[b, :, :].astype(
                jnp.float32
            )

    return pl.pallas_call(
        body,
        out_shape=jax.ShapeDtypeStruct((B, S_HALF, N), jnp.float32),
        in_specs=[
            pl.BlockSpec(memory_space=pltpu.VMEM),
            pl.BlockSpec(memory_space=pltpu.VMEM),
        ],
        out_specs=pl.BlockSpec(memory_space=pltpu.VMEM),
        scratch_shapes=[
            pltpu.VMEM((B, S_HALF, N), jnp.bfloat16),
            pltpu.VMEM((B, S_HALF, N), jnp.bfloat16),
            pltpu.SemaphoreType.DMA,
            pltpu.SemaphoreType.DMA,
        ],
        compiler_params=pltpu.CompilerParams(collective_id=0),
    )(o, w)


# baseline (device time: 315514 ns/iter reference)
import jax
import jax.numpy as jnp
from jax import lax
from jax.experimental import pallas as pl
from jax.experimental.pallas import tpu as pltpu

B = 4
S = 1024
H_SHARD = 16
D = 128
K = H_SHARD * D
N = 4096
S_HALF = S // 2
NT = 2048


def kernel(O, Wo):
    o = O.reshape(B, S, K).astype(jnp.bfloat16)
    w = Wo.astype(jnp.bfloat16)

    def body(
        o_hbm, w_ref, out_hbm,
        o_tile, send_buf, recv_buf, out_stage,
        load_sem, store_sem, send_sem, recv_sems,
    ):
        my_x = lax.axis_index("x")
        my_y = lax.axis_index("y")
        my_z = lax.axis_index("z")
        partner = (my_x, my_y, 1 - my_z)

        barrier_sem = pltpu.get_barrier_semaphore()
        pl.semaphore_signal(
            barrier_sem, inc=1,
            device_id=partner, device_id_type=pl.DeviceIdType.MESH,
        )
        pl.semaphore_wait(barrier_sem, 1)

        send_off = (1 - my_z) * S_HALF
        keep_off = my_z * S_HALF

        rdmas = []
        for b in range(B):
            cp = pltpu.make_async_copy(
                o_hbm.at[b, pl.ds(send_off, S_HALF), :], o_tile, load_sem
            )
            cp.start()
            cp.wait()
            if b >= 1:
                rdmas[b - 1].wait_send()
            for n in range(N // NT):
                p = jnp.dot(
                    o_tile[...], w_ref[:, n * NT:(n + 1) * NT],
                    preferred_element_type=jnp.float32,
                )
                send_buf[:, n * NT:(n + 1) * NT] = p.astype(jnp.bfloat16)
            rdma = pltpu.make_async_remote_copy(
                src_ref=send_buf,
                dst_ref=recv_buf.at[b],
                send_sem=send_sem,
                recv_sem=recv_sems.at[b],
                device_id=partner,
                device_id_type=pl.DeviceIdType.MESH,
            )
            rdma.start()
            rdmas.append(rdma)

        for b in range(B):
            cp = pltpu.make_async_copy(
                o_hbm.at[b, pl.ds(keep_off, S_HALF), :], o_tile, load_sem
            )
            cp.start()
            cp.wait()
            rdmas[b].wait_recv()
            for n in range(N // NT):
                p = jnp.dot(
                    o_tile[...], w_ref[:, n * NT:(n + 1) * NT],
                    preferred_element_type=jnp.float32,
                )
                out_stage[...] = p + recv_buf[
                    b, :, n * NT:(n + 1) * NT
                ].astype(jnp.float32)
                st = pltpu.make_async_copy(
                    out_stage,
                    out_hbm.at[b, :, pl.ds(n * NT, NT)],
                    store_sem,
                )
                st.start()
                st.wait()

        rdmas[B - 1].wait_send()

    return pl.pallas_call(
        body,
        out_shape=jax.ShapeDtypeStruct((B, S_HALF, N), jnp.float32),
        in_specs=[
            pl.BlockSpec(memory_space=pl.ANY),
            pl.BlockSpec(memory_space=pltpu.VMEM),
        ],
        out_specs=pl.BlockSpec(memory_space=pl.ANY),
        scratch_shapes=[
            pltpu.VMEM((S_HALF, K), jnp.bfloat16),
            pltpu.VMEM((S_HALF, N), jnp.bfloat16),
            pltpu.VMEM((B, S_HALF, N), jnp.bfloat16),
            pltpu.VMEM((S_HALF, NT), jnp.float32),
            pltpu.SemaphoreType.DMA,
            pltpu.SemaphoreType.DMA,
            pltpu.SemaphoreType.DMA,
            pltpu.SemaphoreType.DMA((B,)),
        ],
        compiler_params=pltpu.CompilerParams(
            collective_id=0,
            vmem_limit_bytes=44 * 1024 * 1024,
        ),
    )(o, w)
